# baseline (speedup 1.0000x reference)
import functools

import jax
import jax.numpy as jnp
from jax import lax
from jax.experimental import pallas as pl
from jax.experimental.pallas import tpu as pltpu

N = 10000
NP = 10240
E = 160000
EP = 161792
NODE_V = 16
EDGE_V = 8
HID = 256
HALF = 128
LAT = 128
MAXA = 38
NGRAPH = 400
NFLOW = 4
FHID = 256


def _mlp_body(h_ref, a_ref, w1_ref, b1_ref, w2_ref, b2_ref, o_ref):
    hb = jnp.concatenate([h_ref[0] + a_ref[0], h_ref[1] + a_ref[1]], axis=-1)
    t = jnp.maximum(
        jnp.dot(hb, w1_ref[...].T, preferred_element_type=jnp.float32) + b1_ref[...],
        0.0,
    )
    o = jnp.maximum(
        jnp.dot(t, w2_ref[...].T, preferred_element_type=jnp.float32) + b2_ref[...],
        0.0,
    )
    o_ref[0] = o[:, :HALF]
    o_ref[1] = o[:, HALF:]


def _mlp_layer(h2, aggr2, w1, b1, w2, b2):
    rows = 1024
    grid = NP // rows
    return pl.pallas_call(
        _mlp_body,
        grid=(grid,),
        in_specs=[
            pl.BlockSpec((2, rows, HALF), lambda i: (0, i, 0)),
            pl.BlockSpec((2, rows, HALF), lambda i: (0, i, 0)),
            pl.BlockSpec((HID, HID), lambda i: (0, 0)),
            pl.BlockSpec((HID,), lambda i: (0,)),
            pl.BlockSpec((HID, HID), lambda i: (0, 0)),
            pl.BlockSpec((HID,), lambda i: (0,)),
        ],
        out_specs=pl.BlockSpec((2, rows, HALF), lambda i: (0, i, 0)),
        out_shape=jax.ShapeDtypeStruct((2, NP, HALF), jnp.float32),
    )(h2, aggr2, w1, b1, w2, b2)


def _flows_body(hg_ref, eps_ref, muw_ref, mub_ref, lvw_ref, lvb_ref,
                fw1_ref, fb1_ref, fw2_ref, fb2_ref,
                mu_ref, lv_ref, z0_ref, zk_ref, sld_ref):
    hg = jnp.concatenate([hg_ref[0], hg_ref[1]], axis=-1)
    mu = jnp.dot(hg, muw_ref[...].T, preferred_element_type=jnp.float32) + mub_ref[...]
    logvar = jnp.dot(hg, lvw_ref[...].T, preferred_element_type=jnp.float32) + lvb_ref[...]
    std = jnp.exp(0.5 * logvar)
    z = mu + eps_ref[...] * std
    mu_ref[...] = mu
    lv_ref[...] = logvar
    z0_ref[...] = z

    ho1 = lax.broadcasted_iota(jnp.int32, (FHID, LAT), 0) % (LAT - 1) + 1
    io1 = lax.broadcasted_iota(jnp.int32, (FHID, LAT), 1) + 1
    mask1 = (ho1 >= io1).astype(jnp.float32)
    oo2 = lax.broadcasted_iota(jnp.int32, (2 * LAT, FHID), 0) % LAT + 1
    ho2 = lax.broadcasted_iota(jnp.int32, (2 * LAT, FHID), 1) % (LAT - 1) + 1
    mask2 = (oo2 > ho2).astype(jnp.float32)

    sld = jnp.zeros((NGRAPH, 1), jnp.float32)
    for f in range(NFLOW):
        w1m = fw1_ref[f] * mask1
        w2m = fw2_ref[f] * mask2
        hm = jnp.maximum(
            jnp.dot(z, w1m.T, preferred_element_type=jnp.float32) + fb1_ref[f], 0.0
        )
        out = jnp.dot(hm, w2m.T, preferred_element_type=jnp.float32) + fb2_ref[f]
        m = out[:, :LAT]
        gate = jax.nn.sigmoid(out[:, LAT:])
        z = gate * z + (1.0 - gate) * m
        sld = sld + jnp.sum(jnp.log(gate + 1e-08), axis=-1, keepdims=True)
    zk_ref[...] = z
    sld_ref[...] = sld


def _flows(hg2, eps, p):
    fw1 = jnp.stack([f["W1"] for f in p["flows"]])
    fb1 = jnp.stack([f["b1"] for f in p["flows"]])
    fw2 = jnp.stack([f["W2"] for f in p["flows"]])
    fb2 = jnp.stack([f["b2"] for f in p["flows"]])
    full = lambda *s: pl.BlockSpec(s, lambda: tuple(0 for _ in s))
    return pl.pallas_call(
        _flows_body,
        in_specs=[
            full(2, NGRAPH, HALF), full(NGRAPH, LAT),
            full(LAT, HID), full(LAT,), full(LAT, HID), full(LAT,),
            full(NFLOW, FHID, LAT), full(NFLOW, FHID),
            full(NFLOW, 2 * LAT, FHID), full(NFLOW, 2 * LAT),
        ],
        out_specs=[
            full(NGRAPH, LAT), full(NGRAPH, LAT), full(NGRAPH, LAT),
            full(NGRAPH, LAT), full(NGRAPH, 1),
        ],
        out_shape=[
            jax.ShapeDtypeStruct((NGRAPH, LAT), jnp.float32),
            jax.ShapeDtypeStruct((NGRAPH, LAT), jnp.float32),
            jax.ShapeDtypeStruct((NGRAPH, LAT), jnp.float32),
            jax.ShapeDtypeStruct((NGRAPH, LAT), jnp.float32),
            jax.ShapeDtypeStruct((NGRAPH, 1), jnp.float32),
        ],
    )(hg2, eps, p["fc_mu_W"], p["fc_mu_b"], p["fc_lv_W"], p["fc_lv_b"],
      fw1, fb1, fw2, fb2)


_EB = MAXA * EDGE_V


def _dec_body(zk_ref, nw1_ref, nb1_ref, nw2_ref, nb2_ref,
              ew1_ref, eb1_ref, w2a_ref, w2b_ref, bsym_ref,
              node_ref, edge_ref, he_ref):
    i = pl.program_id(0)

    @pl.when(i == 0)
    def _():
        zk = zk_ref[...]
        hn = jnp.maximum(
            jnp.dot(zk, nw1_ref[...].T, preferred_element_type=jnp.float32)
            + nb1_ref[...], 0.0)
        node_ref[...] = (
            jnp.dot(hn, nw2_ref[...].T, preferred_element_type=jnp.float32)
            + nb2_ref[...])
        he_ref[...] = jnp.maximum(
            jnp.dot(zk, ew1_ref[...].T, preferred_element_type=jnp.float32)
            + eb1_ref[...], 0.0)

    wsym = 0.5 * (w2a_ref[0] + w2b_ref[...].reshape(_EB, 512))
    edge_ref[0] = (
        jnp.dot(he_ref[...], wsym.T, preferred_element_type=jnp.float32)
        + bsym_ref[0, 0])


def _decoder(zk, p):
    w2r1 = p["de_W2"].reshape(MAXA, _EB, 512)
    w2r2 = p["de_W2"].reshape(MAXA, MAXA, EDGE_V, 512)
    b2r = p["de_b2"].reshape(MAXA, MAXA, EDGE_V)
    bsym = (0.5 * (b2r + b2r.transpose(1, 0, 2))).reshape(MAXA, 1, _EB)
    node, edge = pl.pallas_call(
        _dec_body,
        grid=(MAXA,),
        in_specs=[
            pl.BlockSpec((NGRAPH, LAT), lambda i: (0, 0)),
            pl.BlockSpec((256, LAT), lambda i: (0, 0)),
            pl.BlockSpec((256,), lambda i: (0,)),
            pl.BlockSpec((MAXA * NODE_V, 256), lambda i: (0, 0)),
            pl.BlockSpec((MAXA * NODE_V,), lambda i: (0,)),
            pl.BlockSpec((512, LAT), lambda i: (0, 0)),
            pl.BlockSpec((512,), lambda i: (0,)),
            pl.BlockSpec((1, _EB, 512), lambda i: (i, 0, 0)),
            pl.BlockSpec((MAXA, 1, EDGE_V, 512), lambda i: (0, i, 0, 0)),
            pl.BlockSpec((1, 1, _EB), lambda i: (i, 0, 0)),
        ],
        out_specs=[
            pl.BlockSpec((NGRAPH, MAXA * NODE_V), lambda i: (0, 0)),
            pl.BlockSpec((1, NGRAPH, _EB), lambda i: (i, 0, 0)),
        ],
        out_shape=[
            jax.ShapeDtypeStruct((NGRAPH, MAXA * NODE_V), jnp.float32),
            jax.ShapeDtypeStruct((MAXA, NGRAPH, _EB), jnp.float32),
        ],
        scratch_shapes=[pltpu.VMEM((NGRAPH, 512), jnp.float32)],
    )(zk, p["dn_W1"], p["dn_b1"], p["dn_W2"], p["dn_b2"],
      p["de_W1"], p["de_b1"], w2r1, w2r2, bsym)
    return node, edge


def _to2(a):
    ap = jnp.concatenate([a, jnp.zeros((NP - N, HID), jnp.float32)], axis=0)
    return jnp.stack([ap[:, :HALF], ap[:, HALF:]])


def _from2(h2):
    return jnp.concatenate([h2[0, :N], h2[1, :N]], axis=-1)


def _embed_nodes(x, node_emb):
    return _to2(jnp.take(node_emb, x, axis=0))


def _msg_aggr(h2, src, dst, attr, edge_emb):
    h = _from2(h2)
    msg = jnp.maximum(jnp.take(h, src, axis=0) + jnp.take(edge_emb, attr, axis=0), 0.0)
    aggr = jnp.zeros((N, HID), jnp.float32).at[dst].add(msg)
    return _to2(aggr)


def _pool(h2, batch):
    hg = jax.ops.segment_sum(_from2(h2), batch, num_segments=NGRAPH)
    return jnp.stack([hg[:, :HALF], hg[:, HALF:]])


def kernel(x, edge_index, edge_attr, batch, eps, params):
    p = params
    src, dst = edge_index[0], edge_index[1]

    h2 = _embed_nodes(x, p["node_emb"])
    for cp in p["convs"]:
        aggr2 = _msg_aggr(h2, src, dst, edge_attr, p["edge_emb"])
        h2 = _mlp_layer(h2, aggr2, cp["W1"], cp["b1"], cp["W2"], cp["b2"])
    hg2 = _pool(h2, batch)
    mu, logvar, z0, zk, sld = _flows(hg2, eps, p)
    node_flat, edge_a1 = _decoder(zk, p)
    node_logits = node_flat.reshape(NGRAPH, MAXA, NODE_V)
    edge_logits = edge_a1.transpose(1, 0, 2).reshape(NGRAPH, MAXA, MAXA, EDGE_V)
    return (node_logits, edge_logits, mu, logvar, z0, zk, sld.reshape(NGRAPH))


# speedup vs baseline: 1.3180x; 1.3180x over previous
import functools

import jax
import jax.numpy as jnp
from jax import lax
from jax.experimental import pallas as pl
from jax.experimental.pallas import tpu as pltpu
from jax.experimental.pallas import tpu_sc as plsc

N = 10000
NP = 10240
E = 160000
EP = 161792
NODE_V = 16
EDGE_V = 8
HID = 256
HALF = 128
LAT = 128
MAXA = 38
NGRAPH = 400
NFLOW = 4
FHID = 256


def _mlp_body(h_ref, a_ref, w1_ref, b1_ref, w2_ref, b2_ref, o_ref):
    hb = jnp.concatenate([h_ref[0] + a_ref[0], h_ref[1] + a_ref[1]], axis=-1)
    t = jnp.maximum(
        jnp.dot(hb, w1_ref[...].T, preferred_element_type=jnp.float32) + b1_ref[...],
        0.0,
    )
    o = jnp.maximum(
        jnp.dot(t, w2_ref[...].T, preferred_element_type=jnp.float32) + b2_ref[...],
        0.0,
    )
    o_ref[0] = o[:, :HALF]
    o_ref[1] = o[:, HALF:]


def _mlp_layer(h2, aggr2, w1, b1, w2, b2):
    rows = 1024
    grid = NP // rows
    return pl.pallas_call(
        _mlp_body,
        grid=(grid,),
        in_specs=[
            pl.BlockSpec((2, rows, HALF), lambda i: (0, i, 0)),
            pl.BlockSpec((2, rows, HALF), lambda i: (0, i, 0)),
            pl.BlockSpec((HID, HID), lambda i: (0, 0)),
            pl.BlockSpec((HID,), lambda i: (0,)),
            pl.BlockSpec((HID, HID), lambda i: (0, 0)),
            pl.BlockSpec((HID,), lambda i: (0,)),
        ],
        out_specs=pl.BlockSpec((2, rows, HALF), lambda i: (0, i, 0)),
        out_shape=jax.ShapeDtypeStruct((2, NP, HALF), jnp.float32),
    )(h2, aggr2, w1, b1, w2, b2)


def _flows_body(hg_ref, eps_ref, muw_ref, mub_ref, lvw_ref, lvb_ref,
                fw1_ref, fb1_ref, fw2_ref, fb2_ref,
                mu_ref, lv_ref, z0_ref, zk_ref, sld_ref):
    hg = jnp.concatenate([hg_ref[0], hg_ref[1]], axis=-1)
    mu = jnp.dot(hg, muw_ref[...].T, preferred_element_type=jnp.float32) + mub_ref[...]
    logvar = jnp.dot(hg, lvw_ref[...].T, preferred_element_type=jnp.float32) + lvb_ref[...]
    std = jnp.exp(0.5 * logvar)
    z = mu + eps_ref[...] * std
    mu_ref[...] = mu
    lv_ref[...] = logvar
    z0_ref[...] = z

    ho1 = lax.broadcasted_iota(jnp.int32, (FHID, LAT), 0) % (LAT - 1) + 1
    io1 = lax.broadcasted_iota(jnp.int32, (FHID, LAT), 1) + 1
    mask1 = (ho1 >= io1).astype(jnp.float32)
    oo2 = lax.broadcasted_iota(jnp.int32, (2 * LAT, FHID), 0) % LAT + 1
    ho2 = lax.broadcasted_iota(jnp.int32, (2 * LAT, FHID), 1) % (LAT - 1) + 1
    mask2 = (oo2 > ho2).astype(jnp.float32)

    sld = jnp.zeros((NGRAPH, 1), jnp.float32)
    for f in range(NFLOW):
        w1m = fw1_ref[f] * mask1
        w2m = fw2_ref[f] * mask2
        hm = jnp.maximum(
            jnp.dot(z, w1m.T, preferred_element_type=jnp.float32) + fb1_ref[f], 0.0
        )
        out = jnp.dot(hm, w2m.T, preferred_element_type=jnp.float32) + fb2_ref[f]
        m = out[:, :LAT]
        gate = jax.nn.sigmoid(out[:, LAT:])
        z = gate * z + (1.0 - gate) * m
        sld = sld + jnp.sum(jnp.log(gate + 1e-08), axis=-1, keepdims=True)
    zk_ref[...] = z
    sld_ref[...] = sld


def _flows(hg2, eps, p):
    fw1 = jnp.stack([f["W1"] for f in p["flows"]])
    fb1 = jnp.stack([f["b1"] for f in p["flows"]])
    fw2 = jnp.stack([f["W2"] for f in p["flows"]])
    fb2 = jnp.stack([f["b2"] for f in p["flows"]])
    full = lambda *s: pl.BlockSpec(s, lambda: tuple(0 for _ in s))
    return pl.pallas_call(
        _flows_body,
        in_specs=[
            full(2, NGRAPH, HALF), full(NGRAPH, LAT),
            full(LAT, HID), full(LAT,), full(LAT, HID), full(LAT,),
            full(NFLOW, FHID, LAT), full(NFLOW, FHID),
            full(NFLOW, 2 * LAT, FHID), full(NFLOW, 2 * LAT),
        ],
        out_specs=[
            full(NGRAPH, LAT), full(NGRAPH, LAT), full(NGRAPH, LAT),
            full(NGRAPH, LAT), full(NGRAPH, 1),
        ],
        out_shape=[
            jax.ShapeDtypeStruct((NGRAPH, LAT), jnp.float32),
            jax.ShapeDtypeStruct((NGRAPH, LAT), jnp.float32),
            jax.ShapeDtypeStruct((NGRAPH, LAT), jnp.float32),
            jax.ShapeDtypeStruct((NGRAPH, LAT), jnp.float32),
            jax.ShapeDtypeStruct((NGRAPH, 1), jnp.float32),
        ],
    )(hg2, eps, p["fc_mu_W"], p["fc_mu_b"], p["fc_lv_W"], p["fc_lv_b"],
      fw1, fb1, fw2, fb2)


_EB = MAXA * EDGE_V


def _dec_body(zk_ref, nw1_ref, nb1_ref, nw2_ref, nb2_ref,
              ew1_ref, eb1_ref, w2a_ref, w2b_ref, bsym_ref,
              node_ref, edge_ref, he_ref):
    i = pl.program_id(0)

    @pl.when(i == 0)
    def _():
        zk = zk_ref[...]
        hn = jnp.maximum(
            jnp.dot(zk, nw1_ref[...].T, preferred_element_type=jnp.float32)
            + nb1_ref[...], 0.0)
        node_ref[...] = (
            jnp.dot(hn, nw2_ref[...].T, preferred_element_type=jnp.float32)
            + nb2_ref[...])
        he_ref[...] = jnp.maximum(
            jnp.dot(zk, ew1_ref[...].T, preferred_element_type=jnp.float32)
            + eb1_ref[...], 0.0)

    wsym = 0.5 * (w2a_ref[0] + w2b_ref[...].reshape(_EB, 512))
    edge_ref[0] = (
        jnp.dot(he_ref[...], wsym.T, preferred_element_type=jnp.float32)
        + bsym_ref[0, 0])


def _decoder(zk, p):
    w2r1 = p["de_W2"].reshape(MAXA, _EB, 512)
    w2r2 = p["de_W2"].reshape(MAXA, MAXA, EDGE_V, 512)
    b2r = p["de_b2"].reshape(MAXA, MAXA, EDGE_V)
    bsym = (0.5 * (b2r + b2r.transpose(1, 0, 2))).reshape(MAXA, 1, _EB)
    node, edge = pl.pallas_call(
        _dec_body,
        grid=(MAXA,),
        in_specs=[
            pl.BlockSpec((NGRAPH, LAT), lambda i: (0, 0)),
            pl.BlockSpec((256, LAT), lambda i: (0, 0)),
            pl.BlockSpec((256,), lambda i: (0,)),
            pl.BlockSpec((MAXA * NODE_V, 256), lambda i: (0, 0)),
            pl.BlockSpec((MAXA * NODE_V,), lambda i: (0,)),
            pl.BlockSpec((512, LAT), lambda i: (0, 0)),
            pl.BlockSpec((512,), lambda i: (0,)),
            pl.BlockSpec((1, _EB, 512), lambda i: (i, 0, 0)),
            pl.BlockSpec((MAXA, 1, EDGE_V, 512), lambda i: (0, i, 0, 0)),
            pl.BlockSpec((1, 1, _EB), lambda i: (i, 0, 0)),
        ],
        out_specs=[
            pl.BlockSpec((NGRAPH, MAXA * NODE_V), lambda i: (0, 0)),
            pl.BlockSpec((1, NGRAPH, _EB), lambda i: (i, 0, 0)),
        ],
        out_shape=[
            jax.ShapeDtypeStruct((NGRAPH, MAXA * NODE_V), jnp.float32),
            jax.ShapeDtypeStruct((MAXA, NGRAPH, _EB), jnp.float32),
        ],
        scratch_shapes=[pltpu.VMEM((NGRAPH, 512), jnp.float32)],
    )(zk, p["dn_W1"], p["dn_b1"], p["dn_W2"], p["dn_b2"],
      p["de_W1"], p["de_b1"], w2r1, w2r2, bsym)
    return node, edge


NTILE = 16
CHUNK = 128
NCH = EP // (NTILE * CHUNK)
STRIPE = NP // NTILE
NXCH = STRIPE // CHUNK
_SC_MESH = plsc.VectorSubcoreMesh(core_axis_name="c", subcore_axis_name="s")


def _zero_fill(zero_v):
    def zrow(r, _):
        for j in range(HALF // 16):
            zero_v[r, pl.ds(j * 16, 16)] = jnp.zeros((16,), jnp.float32)
        return 0

    lax.fori_loop(0, CHUNK, zrow, 0)


@functools.partial(
    pl.kernel, mesh=_SC_MESH,
    out_type=jax.ShapeDtypeStruct((2, NP, HALF), jnp.float32),
    scratch_types=[
        pltpu.VMEM((NXCH, CHUNK), jnp.int32),
        pltpu.VMEM((CHUNK, HALF), jnp.float32),
        pltpu.SemaphoreType.DMA,
    ],
)
def _sc_embed(emb_hbm, x_hbm, out_hbm, idx_v, rows_v, sem):
    c = lax.axis_index("c")
    s = lax.axis_index("s")
    pltpu.sync_copy(x_hbm.at[s], idx_v)

    def body(i, _):
        pltpu.async_copy(emb_hbm.at[c].at[idx_v.at[i]], rows_v, sem).wait()
        pltpu.sync_copy(rows_v, out_hbm.at[c].at[pl.ds(s * STRIPE + i * CHUNK, CHUNK)])
        return 0

    lax.fori_loop(0, NXCH, body, 0)


@functools.partial(
    pl.kernel, mesh=_SC_MESH,
    out_type=jax.ShapeDtypeStruct((2, NP, HALF), jnp.float32),
    scratch_types=[
        pltpu.VMEM((CHUNK,), jnp.int32),
        pltpu.VMEM((CHUNK,), jnp.int32),
        pltpu.VMEM((CHUNK,), jnp.int32),
        pltpu.VMEM((CHUNK, HALF), jnp.float32),
        pltpu.VMEM((CHUNK, HALF), jnp.float32),
        pltpu.VMEM_SHARED((NP, HALF), jnp.float32),
        pltpu.SemaphoreType.DMA,
        pltpu.SemaphoreType.DMA,
    ],
)
def _sc_msg(h_hbm, src_hbm, dst_hbm, attr_hbm, emb_hbm, out_hbm,
            src_v, dst_v, attr_v, rows_v, ee_v, aggr_sh, sem1, sem2):
    c = lax.axis_index("c")
    s = lax.axis_index("s")
    _zero_fill(ee_v)
    for i in range(STRIPE // CHUNK):
        pltpu.sync_copy(ee_v, aggr_sh.at[pl.ds(s * STRIPE + i * CHUNK, CHUNK)])
    plsc.subcore_barrier()

    def body(i, _):
        pltpu.sync_copy(src_hbm.at[s].at[i], src_v)
        pltpu.sync_copy(attr_hbm.at[s].at[i], attr_v)
        g1 = pltpu.async_copy(h_hbm.at[c].at[src_v], rows_v, sem1)
        g2 = pltpu.async_copy(emb_hbm.at[c].at[attr_v], ee_v, sem2)
        pltpu.sync_copy(dst_hbm.at[s].at[i], dst_v)
        g1.wait()
        g2.wait()

        def crow(r, _):
            for j in range(HALF // 16):
                sl = pl.ds(j * 16, 16)
                rows_v[r, sl] = jnp.maximum(rows_v[r, sl] + ee_v[r, sl], 0.0)
            return 0

        lax.fori_loop(0, CHUNK, crow, 0)
        pltpu.sync_copy(rows_v, aggr_sh.at[dst_v], add=True)
        return 0

    lax.fori_loop(0, NCH, body, 0)
    plsc.subcore_barrier()
    pltpu.sync_copy(aggr_sh.at[pl.ds(s * STRIPE, STRIPE)],
                    out_hbm.at[c].at[pl.ds(s * STRIPE, STRIPE)])


NG_PAD = 512
GSTRIPE = NG_PAD // NTILE


@functools.partial(
    pl.kernel, mesh=_SC_MESH,
    out_type=jax.ShapeDtypeStruct((2, NG_PAD, HALF), jnp.float32),
    scratch_types=[
        pltpu.VMEM((NXCH, CHUNK), jnp.int32),
        pltpu.VMEM((CHUNK, HALF), jnp.float32),
        pltpu.VMEM((GSTRIPE, HALF), jnp.float32),
        pltpu.VMEM_SHARED((NG_PAD, HALF), jnp.float32),
    ],
)
def _sc_pool(h_hbm, batch_hbm, out_hbm, idx_v, rows_v, zero_v, hg_sh):
    c = lax.axis_index("c")
    s = lax.axis_index("s")

    def zrow(r, _):
        for j in range(HALF // 16):
            zero_v[r, pl.ds(j * 16, 16)] = jnp.zeros((16,), jnp.float32)
        return 0

    lax.fori_loop(0, GSTRIPE, zrow, 0)
    pltpu.sync_copy(zero_v, hg_sh.at[pl.ds(s * GSTRIPE, GSTRIPE)])
    pltpu.sync_copy(batch_hbm.at[s], idx_v)
    plsc.subcore_barrier()

    def body(i, _):
        pltpu.sync_copy(h_hbm.at[c].at[pl.ds(s * STRIPE + i * CHUNK, CHUNK)], rows_v)
        pltpu.sync_copy(rows_v, hg_sh.at[idx_v.at[i]], add=True)
        return 0

    lax.fori_loop(0, NXCH, body, 0)
    plsc.subcore_barrier()
    pltpu.sync_copy(hg_sh.at[pl.ds(s * GSTRIPE, GSTRIPE)],
                    out_hbm.at[c].at[pl.ds(s * GSTRIPE, GSTRIPE)])


def _embed_nodes(xT, node_emb2):
    return _sc_embed(node_emb2, xT)


def _msg_aggr(h2, srcT, dstT, attrT, edge_emb2):
    return _sc_msg(h2, srcT, dstT, attrT, edge_emb2)


def _pool(h2, batchT):
    return _sc_pool(h2, batchT)[:, :NGRAPH]


def kernel(x, edge_index, edge_attr, batch, eps, params):
    p = params
    src, dst = edge_index[0], edge_index[1]
    node_emb2 = p["node_emb"].reshape(NODE_V, 2, HALF).transpose(1, 0, 2)
    edge_emb2 = p["edge_emb"].reshape(EDGE_V, 2, HALF).transpose(1, 0, 2)
    srcT = jnp.pad(src, (0, EP - E)).reshape(NTILE, NCH, CHUNK)
    dstT = jnp.pad(dst, (0, EP - E), constant_values=N).reshape(NTILE, NCH, CHUNK)
    attrT = jnp.pad(edge_attr, (0, EP - E)).reshape(NTILE, NCH, CHUNK)
    xT = jnp.pad(x, (0, NP - N)).reshape(NTILE, NXCH, CHUNK)
    batchT = jnp.pad(batch, (0, NP - N),
                     constant_values=NGRAPH).reshape(NTILE, NXCH, CHUNK)

    h2 = _embed_nodes(xT, node_emb2)
    for cp in p["convs"]:
        aggr2 = _msg_aggr(h2, srcT, dstT, attrT, edge_emb2)
        h2 = _mlp_layer(h2, aggr2, cp["W1"], cp["b1"], cp["W2"], cp["b2"])
    hg2 = _pool(h2, batchT)
    mu, logvar, z0, zk, sld = _flows(hg2, eps, p)
    node_flat, edge_a1 = _decoder(zk, p)
    node_logits = node_flat.reshape(NGRAPH, MAXA, NODE_V)
    edge_logits = edge_a1.transpose(1, 0, 2).reshape(NGRAPH, MAXA, MAXA, EDGE_V)
    return (node_logits, edge_logits, mu, logvar, z0, zk, sld.reshape(NGRAPH))
